# baseline (speedup 1.0000x reference)
import functools

import jax
import jax.numpy as jnp
from jax import lax
from jax.experimental import pallas as pl
from jax.experimental.pallas import tpu as pltpu
from jax.experimental.pallas import tpu_sc as plsc

_info = plsc.get_sparse_core_info()
_NC, _NS = _info.num_cores, _info.num_subcores
_NW = _NC * _NS

_V = 1_000_000
_D = 64
_V0 = 999_936
_NB = 7812
_NG1 = 246
_BATCH = 16384
_HIST = 20
_B = _BATCH * _HIST
_BPW = _B // _NW
_BB = 512
_NQ = 4 * _HIST
_NGRP = _NQ // 8

_mesh = plsc.VectorSubcoreMesh(core_axis_name="c", subcore_axis_name="s")
_i32 = jnp.int32


def _iota16():
    return lax.iota(_i32, 16)


@functools.partial(
    pl.kernel,
    mesh=_mesh,
    out_type=jax.ShapeDtypeStruct((_V // 2, 128), jnp.float32),
    scratch_types=[
        pltpu.VMEM((2, 8, 8, 128), jnp.float32),
        pltpu.VMEM((2, 64, 128), jnp.float32),
        pltpu.VMEM((32, 128), jnp.float32),
        pltpu.SemaphoreType.DMA,
        pltpu.SemaphoreType.DMA,
        pltpu.SemaphoreType.DMA,
        pltpu.SemaphoreType.DMA,
    ],
    compiler_params=pltpu.CompilerParams(
        use_tc_tiling_on_sc=True, needs_layout_passes=False),
)
def _s1(tT_hbm, tail_hbm, out_hbm, inb, outb, tailb, sia, sib, soa, sob):
    wid = lax.axis_index("s") * _NC + lax.axis_index("c")
    it = _iota16()

    pltpu.sync_copy(tail_hbm, tailb)
    pltpu.sync_copy(tailb, out_hbm.at[pl.ds(_V0 // 2, 32)])

    def blk(g):
        return jnp.minimum(wid + _NW * g, _NB - 1)

    def fire_in(g, buf, sem):
        ib = blk(g)
        c0 = pl.multiple_of(ib * 128, 128)
        hs = []
        for jt in range(8):
            hs.append(pltpu.make_async_copy(
                tT_hbm.at[pl.ds(8 * jt, 8), pl.ds(c0, 128)],
                inb.at[buf, jt], sem))
        for h in hs:
            h.start()
        return hs

    def wait_in(g, buf, sem):
        ib = blk(g)
        c0 = pl.multiple_of(ib * 128, 128)
        for jt in range(8):
            pltpu.make_async_copy(
                tT_hbm.at[pl.ds(8 * jt, 8), pl.ds(c0, 128)],
                inb.at[buf, jt], sem).wait()

    def out_slice(g):
        r0 = pl.multiple_of(blk(g) * 64, 8)
        return out_hbm.at[pl.ds(r0, 64)]

    jt_vecs = [2 * m + (it >> 3) for m in range(4)]
    js_vec = it & 7

    def transpose_block(buf):
        src = inb.at[buf]

        def row(r, carry):
            for lb in range(8):
                p = lb // 4
                cl = jnp.full((16,), 2 * r + p, _i32)
                v = plsc.load_gather(src, [jt_vecs[lb % 4], js_vec, cl])
                outb[buf, r, pl.ds(16 * lb, 16)] = v
            return carry

        lax.fori_loop(0, 64, row, 0)

    pltpu.make_async_copy(out_hbm.at[pl.ds(0, 64)], outb.at[0], soa).start()
    pltpu.make_async_copy(out_hbm.at[pl.ds(0, 64)], outb.at[1], sob).start()
    fire_in(0, 0, sia)
    fire_in(1, 1, sib)

    def body(k, carry):
        g0 = 2 * k
        wait_in(g0, 0, sia)
        pltpu.make_async_copy(out_hbm.at[pl.ds(0, 64)], outb.at[0], soa).wait()
        transpose_block(0)
        pltpu.make_async_copy(outb.at[0], out_slice(g0), soa).start()
        fire_in(g0 + 2, 0, sia)
        g1 = g0 + 1
        wait_in(g1, 1, sib)
        pltpu.make_async_copy(out_hbm.at[pl.ds(0, 64)], outb.at[1], sob).wait()
        transpose_block(1)
        pltpu.make_async_copy(outb.at[1], out_slice(g1), sob).start()
        fire_in(g1 + 2, 1, sib)
        return carry

    lax.fori_loop(0, _NG1 // 2, body, 0)

    wait_in(_NG1, 0, sia)
    wait_in(_NG1 + 1, 1, sib)
    pltpu.make_async_copy(outb.at[0], out_slice(_NG1 - 2), soa).wait()
    pltpu.make_async_copy(outb.at[1], out_slice(_NG1 - 1), sob).wait()


@functools.partial(
    pl.kernel,
    mesh=_mesh,
    out_type=jax.ShapeDtypeStruct((_HIST, 8, 128, 8, 128), jnp.float32),
    scratch_types=[
        pltpu.VMEM((_BPW,), _i32),
        pltpu.VMEM((_NQ, 128), _i32),
        pltpu.VMEM((8, 128, _D), jnp.float32),
        pltpu.VMEM((4, _D, 128), jnp.float32),
        pltpu.SemaphoreType.DMA,
    ]
    + [pltpu.SemaphoreType.DMA] * 8
    + [pltpu.SemaphoreType.DMA] * 4,
    compiler_params=pltpu.CompilerParams(
        use_tc_tiling_on_sc=False, needs_layout_passes=False),
)
def _s2(idx_hbm, table_hbm, out_hbm, idx_v, icols, rows_v, tbuf, sidx, *sems):
    sem_g = sems[:8]
    sem_o = sems[8:]
    wid = lax.axis_index("s") * _NC + lax.axis_index("c")
    base = wid * _BPW
    it = _iota16()

    pltpu.async_copy(idx_hbm.at[pl.ds(base, _BPW)], idx_v, sidx).wait()

    def extract(q, carry):
        b1 = q // _HIST
        h = q % _HIST
        for m in range(8):
            fv = (128 * b1 + 16 * m + it) * _HIST + h
            v = plsc.load_gather(idx_v, [fv])
            icols[q, pl.ds(16 * m, 16)] = v
        return carry

    lax.fori_loop(0, _NQ, extract, 0)

    jrow_vecs = [16 * jb + it for jb in range(4)]

    def group(grp, carry):
        hg = []
        for j8 in range(8):
            q = grp * 8 + j8
            hg.append(pltpu.async_copy(
                table_hbm.at[icols.at[q]], rows_v.at[j8], sem_g[j8]))
        ho = []
        for j8 in range(8):
            q = grp * 8 + j8
            b1 = q // _HIST + 4 * wid
            h = q % _HIST
            tb = j8 % 4
            hg[j8].wait()
            if j8 >= 4:
                for hh in ho[(j8 - 4) * 8:(j8 - 3) * 8]:
                    hh.wait()

            def row(l, carry2):
                lv = jnp.full((16,), l, _i32)
                for jb in range(4):
                    v = rows_v[j8, l, pl.ds(16 * jb, 16)]
                    plsc.store_scatter(tbuf.at[tb], [jrow_vecs[jb], lv], v)
                return carry2

            lax.fori_loop(0, 128, row, 0)
            for jg in range(8):
                ho.append(pltpu.async_copy(
                    tbuf.at[tb, pl.ds(8 * jg, 8)],
                    out_hbm.at[h, jg, b1], sem_o[tb]))
        for hh in ho[4 * 8:]:
            hh.wait()
        return carry

    lax.fori_loop(0, _NGRP, group, 0)


def kernel(input_index, embeds):
    tT = embeds.T
    tail = embeds[_V0:].reshape(32, 128)
    tlin = _s1(tT, tail)
    table = tlin.reshape(_V, _D)
    flat_idx = input_index.reshape(-1).astype(_i32)
    out5 = _s2(flat_idx, table)
    out = out5.transpose(2, 4, 0, 1, 3).reshape(_BATCH, _HIST, _D)
    return out


# speedup vs baseline: 1.7136x; 1.7136x over previous
import functools

import jax
import jax.numpy as jnp
from jax import lax
from jax.experimental import pallas as pl
from jax.experimental.pallas import tpu as pltpu
from jax.experimental.pallas import tpu_sc as plsc

_info = plsc.get_sparse_core_info()
_NC, _NS = _info.num_cores, _info.num_subcores
_NW = _NC * _NS

_V = 1_000_000
_D = 64
_V0 = 999_936
_NB = 7812
_NG1 = 246
_BATCH = 16384
_HIST = 20
_B = _BATCH * _HIST
_BPW = _B // _NW
_BB = 512
_NQ = 4 * _HIST
_NGRP = _NQ // 8

_mesh = plsc.VectorSubcoreMesh(core_axis_name="c", subcore_axis_name="s")
_i32 = jnp.int32


def _iota16():
    return lax.iota(_i32, 16)


@functools.partial(
    pl.kernel,
    mesh=_mesh,
    out_type=jax.ShapeDtypeStruct((_V // 2, 128), jnp.float32),
    scratch_types=[
        pltpu.VMEM((2, 8, 8, 128), jnp.float32),
        pltpu.VMEM((2, 64, 128), jnp.float32),
        pltpu.VMEM((32, 128), jnp.float32),
        pltpu.SemaphoreType.DMA,
        pltpu.SemaphoreType.DMA,
        pltpu.SemaphoreType.DMA,
        pltpu.SemaphoreType.DMA,
    ],
    compiler_params=pltpu.CompilerParams(
        use_tc_tiling_on_sc=True, needs_layout_passes=False),
)
def _s1(tT_hbm, tail_hbm, out_hbm, inb, outb, tailb, sia, sib, soa, sob):
    wid = lax.axis_index("s") * _NC + lax.axis_index("c")
    it = _iota16()

    pltpu.sync_copy(tail_hbm, tailb)
    pltpu.sync_copy(tailb, out_hbm.at[pl.ds(_V0 // 2, 32)])

    def blk(g):
        return jnp.minimum(wid + _NW * g, _NB - 1)

    def fire_in(g, buf, sem):
        ib = blk(g)
        c0 = pl.multiple_of(ib * 128, 128)
        hs = []
        for jt in range(8):
            hs.append(pltpu.make_async_copy(
                tT_hbm.at[pl.ds(8 * jt, 8), pl.ds(c0, 128)],
                inb.at[buf, jt], sem))
        for h in hs:
            h.start()
        return hs

    def wait_in(g, buf, sem):
        ib = blk(g)
        c0 = pl.multiple_of(ib * 128, 128)
        for jt in range(8):
            pltpu.make_async_copy(
                tT_hbm.at[pl.ds(8 * jt, 8), pl.ds(c0, 128)],
                inb.at[buf, jt], sem).wait()

    def out_slice(g):
        r0 = pl.multiple_of(blk(g) * 64, 8)
        return out_hbm.at[pl.ds(r0, 64)]

    jt_vecs = [2 * m + (it >> 3) for m in range(4)]
    js_vec = it & 7

    def transpose_block(buf):
        src = inb.at[buf]

        @plsc.parallel_loop(0, 64, unroll=4)
        def _row(r):
            for lb in range(8):
                p = lb // 4
                cl = jnp.full((16,), 2 * r + p, _i32)
                v = plsc.load_gather(src, [jt_vecs[lb % 4], js_vec, cl])
                outb[buf, r, pl.ds(16 * lb, 16)] = v

    pltpu.make_async_copy(out_hbm.at[pl.ds(0, 64)], outb.at[0], soa).start()
    pltpu.make_async_copy(out_hbm.at[pl.ds(0, 64)], outb.at[1], sob).start()
    fire_in(0, 0, sia)
    fire_in(1, 1, sib)

    def body(k, carry):
        g0 = 2 * k
        wait_in(g0, 0, sia)
        pltpu.make_async_copy(out_hbm.at[pl.ds(0, 64)], outb.at[0], soa).wait()
        transpose_block(0)
        pltpu.make_async_copy(outb.at[0], out_slice(g0), soa).start()
        fire_in(g0 + 2, 0, sia)
        g1 = g0 + 1
        wait_in(g1, 1, sib)
        pltpu.make_async_copy(out_hbm.at[pl.ds(0, 64)], outb.at[1], sob).wait()
        transpose_block(1)
        pltpu.make_async_copy(outb.at[1], out_slice(g1), sob).start()
        fire_in(g1 + 2, 1, sib)
        return carry

    lax.fori_loop(0, _NG1 // 2, body, 0)

    wait_in(_NG1, 0, sia)
    wait_in(_NG1 + 1, 1, sib)
    pltpu.make_async_copy(outb.at[0], out_slice(_NG1 - 2), soa).wait()
    pltpu.make_async_copy(outb.at[1], out_slice(_NG1 - 1), sob).wait()


@functools.partial(
    pl.kernel,
    mesh=_mesh,
    out_type=jax.ShapeDtypeStruct((_HIST, 8, 128, 8, 128), jnp.float32),
    scratch_types=[
        pltpu.VMEM((_BPW,), _i32),
        pltpu.VMEM((_NQ, 128), _i32),
        pltpu.VMEM((8, 128, _D), jnp.float32),
        pltpu.VMEM((4, _D, 128), jnp.float32),
        pltpu.SemaphoreType.DMA,
    ]
    + [pltpu.SemaphoreType.DMA] * 8
    + [pltpu.SemaphoreType.DMA] * 4,
    compiler_params=pltpu.CompilerParams(
        use_tc_tiling_on_sc=False, needs_layout_passes=False),
)
def _s2(idx_hbm, table_hbm, out_hbm, idx_v, icols, rows_v, tbuf, sidx, *sems):
    sem_g = sems[:8]
    sem_o = sems[8:]
    wid = lax.axis_index("s") * _NC + lax.axis_index("c")
    base = wid * _BPW
    it = _iota16()

    pltpu.async_copy(idx_hbm.at[pl.ds(base, _BPW)], idx_v, sidx).wait()

    def extract(q, carry):
        b1 = q // _HIST
        h = q % _HIST
        for m in range(8):
            fv = (128 * b1 + 16 * m + it) * _HIST + h
            v = plsc.load_gather(idx_v, [fv])
            icols[q, pl.ds(16 * m, 16)] = v
        return carry

    lax.fori_loop(0, _NQ, extract, 0)

    jrow_vecs = [16 * jb + it for jb in range(4)]

    def group(grp, carry):
        hg = []
        for j8 in range(8):
            q = grp * 8 + j8
            hg.append(pltpu.async_copy(
                table_hbm.at[icols.at[q]], rows_v.at[j8], sem_g[j8]))
        ho = []
        for j8 in range(8):
            q = grp * 8 + j8
            b1 = q // _HIST + 4 * wid
            h = q % _HIST
            tb = j8 % 4
            hg[j8].wait()
            if j8 >= 4:
                for hh in ho[(j8 - 4) * 8:(j8 - 3) * 8]:
                    hh.wait()

            @plsc.parallel_loop(0, 128, unroll=4)
            def _row(l):
                lv = jnp.full((16,), l, _i32)
                for jb in range(4):
                    v = rows_v[j8, l, pl.ds(16 * jb, 16)]
                    plsc.store_scatter(tbuf.at[tb], [jrow_vecs[jb], lv], v)
            for jg in range(8):
                ho.append(pltpu.async_copy(
                    tbuf.at[tb, pl.ds(8 * jg, 8)],
                    out_hbm.at[h, jg, b1], sem_o[tb]))
        for hh in ho[4 * 8:]:
            hh.wait()
        return carry

    lax.fori_loop(0, _NGRP, group, 0)


def kernel(input_index, embeds):
    tT = embeds.T
    tail = embeds[_V0:].reshape(32, 128)
    tlin = _s1(tT, tail)
    table = tlin.reshape(_V, _D)
    flat_idx = input_index.reshape(-1).astype(_i32)
    out5 = _s2(flat_idx, table)
    out = out5.transpose(2, 4, 0, 1, 3).reshape(_BATCH, _HIST, _D)
    return out


# speedup vs baseline: 1.7151x; 1.0009x over previous
import functools

import jax
import jax.numpy as jnp
from jax import lax
from jax.experimental import pallas as pl
from jax.experimental.pallas import tpu as pltpu
from jax.experimental.pallas import tpu_sc as plsc

_info = plsc.get_sparse_core_info()
_NC, _NS = _info.num_cores, _info.num_subcores
_NW = _NC * _NS

_V = 1_000_000
_D = 64
_V0 = 999_936
_NB = 7812
_NG1 = 246
_BATCH = 16384
_HIST = 20
_B = _BATCH * _HIST
_BPW = _B // _NW
_BB = 512
_NQ = 4 * _HIST
_NGRP = _NQ // 8

_mesh = plsc.VectorSubcoreMesh(core_axis_name="c", subcore_axis_name="s")
_i32 = jnp.int32


def _iota16():
    return lax.iota(_i32, 16)


@functools.partial(
    pl.kernel,
    mesh=_mesh,
    out_type=jax.ShapeDtypeStruct((_V // 2, 128), jnp.float32),
    scratch_types=[
        pltpu.VMEM((2, 64, 128), jnp.float32),
        pltpu.VMEM((2, 64, 128), jnp.float32),
        pltpu.VMEM((32, 128), jnp.float32),
        pltpu.SemaphoreType.DMA,
        pltpu.SemaphoreType.DMA,
        pltpu.SemaphoreType.DMA,
        pltpu.SemaphoreType.DMA,
    ],
    compiler_params=pltpu.CompilerParams(
        use_tc_tiling_on_sc=True, needs_layout_passes=False),
)
def _s1(tT_hbm, tail_hbm, out_hbm, inb, outb, tailb, sia, sib, soa, sob):
    wid = lax.axis_index("s") * _NC + lax.axis_index("c")
    it = _iota16()

    pltpu.sync_copy(tail_hbm, tailb)
    pltpu.sync_copy(tailb, out_hbm.at[pl.ds(_V0 // 2, 32)])

    def blk(g):
        return jnp.minimum(wid + _NW * g, _NB - 1)

    def fire_in(g, buf, sem):
        ib = blk(g)
        c0 = pl.multiple_of(ib * 128, 128)
        hs = []
        for jt in range(8):
            hs.append(pltpu.make_async_copy(
                tT_hbm.at[pl.ds(8 * jt, 8), pl.ds(c0, 128)],
                inb.at[buf, pl.ds(8 * jt, 8)], sem))
        for h in hs:
            h.start()
        return hs

    def wait_in(g, buf, sem):
        ib = blk(g)
        c0 = pl.multiple_of(ib * 128, 128)
        for jt in range(8):
            pltpu.make_async_copy(
                tT_hbm.at[pl.ds(8 * jt, 8), pl.ds(c0, 128)],
                inb.at[buf, pl.ds(8 * jt, 8)], sem).wait()

    def out_slice(g):
        r0 = pl.multiple_of(blk(g) * 64, 8)
        return out_hbm.at[pl.ds(r0, 64)]

    j_vecs = [16 * m + it for m in range(4)]

    def transpose_block(buf):
        src = inb.at[buf]

        @plsc.parallel_loop(0, 64, unroll=8)
        def _row(r):
            for lb in range(8):
                p = lb // 4
                cl = jnp.full((16,), 2 * r + p, _i32)
                v = plsc.load_gather(src, [j_vecs[lb % 4], cl])
                outb[buf, r, pl.ds(16 * lb, 16)] = v

    pltpu.make_async_copy(out_hbm.at[pl.ds(0, 64)], outb.at[0], soa).start()
    pltpu.make_async_copy(out_hbm.at[pl.ds(0, 64)], outb.at[1], sob).start()
    fire_in(0, 0, sia)
    fire_in(1, 1, sib)

    def body(k, carry):
        g0 = 2 * k
        wait_in(g0, 0, sia)
        pltpu.make_async_copy(out_hbm.at[pl.ds(0, 64)], outb.at[0], soa).wait()
        transpose_block(0)
        pltpu.make_async_copy(outb.at[0], out_slice(g0), soa).start()
        fire_in(g0 + 2, 0, sia)
        g1 = g0 + 1
        wait_in(g1, 1, sib)
        pltpu.make_async_copy(out_hbm.at[pl.ds(0, 64)], outb.at[1], sob).wait()
        transpose_block(1)
        pltpu.make_async_copy(outb.at[1], out_slice(g1), sob).start()
        fire_in(g1 + 2, 1, sib)
        return carry

    lax.fori_loop(0, _NG1 // 2, body, 0)

    wait_in(_NG1, 0, sia)
    wait_in(_NG1 + 1, 1, sib)
    pltpu.make_async_copy(outb.at[0], out_slice(_NG1 - 2), soa).wait()
    pltpu.make_async_copy(outb.at[1], out_slice(_NG1 - 1), sob).wait()


@functools.partial(
    pl.kernel,
    mesh=_mesh,
    out_type=jax.ShapeDtypeStruct((_HIST, 8, 128, 8, 128), jnp.float32),
    scratch_types=[
        pltpu.VMEM((_BPW,), _i32),
        pltpu.VMEM((_NQ, 128), _i32),
        pltpu.VMEM((8, 128, _D), jnp.float32),
        pltpu.VMEM((4, _D, 128), jnp.float32),
        pltpu.SemaphoreType.DMA,
    ]
    + [pltpu.SemaphoreType.DMA] * 8
    + [pltpu.SemaphoreType.DMA] * 4,
    compiler_params=pltpu.CompilerParams(
        use_tc_tiling_on_sc=False, needs_layout_passes=False),
)
def _s2(idx_hbm, table_hbm, out_hbm, idx_v, icols, rows_v, tbuf, sidx, *sems):
    sem_g = sems[:8]
    sem_o = sems[8:]
    wid = lax.axis_index("s") * _NC + lax.axis_index("c")
    base = wid * _BPW
    it = _iota16()

    pltpu.async_copy(idx_hbm.at[pl.ds(base, _BPW)], idx_v, sidx).wait()

    def extract(q, carry):
        b1 = q // _HIST
        h = q % _HIST
        for m in range(8):
            fv = (128 * b1 + 16 * m + it) * _HIST + h
            v = plsc.load_gather(idx_v, [fv])
            icols[q, pl.ds(16 * m, 16)] = v
        return carry

    lax.fori_loop(0, _NQ, extract, 0)

    jrow_vecs = [16 * jb + it for jb in range(4)]

    def group(grp, carry):
        hg = []
        for j8 in range(8):
            q = grp * 8 + j8
            hg.append(pltpu.async_copy(
                table_hbm.at[icols.at[q]], rows_v.at[j8], sem_g[j8]))
        ho = []
        for j8 in range(8):
            q = grp * 8 + j8
            b1 = q // _HIST + 4 * wid
            h = q % _HIST
            tb = j8 % 4
            hg[j8].wait()
            if j8 >= 4:
                for hh in ho[(j8 - 4) * 8:(j8 - 3) * 8]:
                    hh.wait()

            @plsc.parallel_loop(0, 128, unroll=4)
            def _row(l):
                lv = jnp.full((16,), l, _i32)
                for jb in range(4):
                    v = rows_v[j8, l, pl.ds(16 * jb, 16)]
                    plsc.store_scatter(tbuf.at[tb], [jrow_vecs[jb], lv], v)
            for jg in range(8):
                ho.append(pltpu.async_copy(
                    tbuf.at[tb, pl.ds(8 * jg, 8)],
                    out_hbm.at[h, jg, b1], sem_o[tb]))
        for hh in ho[4 * 8:]:
            hh.wait()
        return carry

    lax.fori_loop(0, _NGRP, group, 0)


def kernel(input_index, embeds):
    tT = embeds.T
    tail = embeds[_V0:].reshape(32, 128)
    tlin = _s1(tT, tail)
    table = tlin.reshape(_V, _D)
    flat_idx = input_index.reshape(-1).astype(_i32)
    out5 = _s2(flat_idx, table)
    out = out5.transpose(2, 4, 0, 1, 3).reshape(_BATCH, _HIST, _D)
    return out


# speedup vs baseline: 2.1590x; 1.2589x over previous
import functools

import jax
import jax.numpy as jnp
from jax import lax
from jax.experimental import pallas as pl
from jax.experimental.pallas import tpu as pltpu
from jax.experimental.pallas import tpu_sc as plsc

_info = plsc.get_sparse_core_info()
_NC, _NS = _info.num_cores, _info.num_subcores
_NW = _NC * _NS

_V = 1_000_000
_D = 64
_V0 = 999_936
_NB = 7812
_NG1 = 246
_BATCH = 16384
_HIST = 20
_B = _BATCH * _HIST
_BPW = _B // _NW
_BB = 512
_NQ = 4 * _HIST
_NGRP = _NQ // 8

_mesh = plsc.VectorSubcoreMesh(core_axis_name="c", subcore_axis_name="s")
_i32 = jnp.int32


def _iota16():
    return lax.iota(_i32, 16)


@functools.partial(
    pl.kernel,
    mesh=_mesh,
    out_type=jax.ShapeDtypeStruct((_V // 2, 128), jnp.float32),
    scratch_types=[
        pltpu.VMEM((2, 64, 131), jnp.float32),
        pltpu.VMEM((2, 64, 128), jnp.float32),
        pltpu.VMEM((32, 128), jnp.float32),
        pltpu.SemaphoreType.DMA,
        pltpu.SemaphoreType.DMA,
        pltpu.SemaphoreType.DMA,
        pltpu.SemaphoreType.DMA,
    ],
    compiler_params=pltpu.CompilerParams(
        use_tc_tiling_on_sc=True, needs_layout_passes=False),
)
def _s1(tT_hbm, tail_hbm, out_hbm, inb, outb, tailb, sia, sib, soa, sob):
    wid = lax.axis_index("s") * _NC + lax.axis_index("c")
    it = _iota16()

    pltpu.sync_copy(tail_hbm, tailb)
    pltpu.sync_copy(tailb, out_hbm.at[pl.ds(_V0 // 2, 32)])

    def blk(g):
        return jnp.minimum(wid + _NW * g, _NB - 1)

    def fire_in(g, buf, sem):
        ib = blk(g)
        c0 = pl.multiple_of(ib * 128, 128)
        hs = []
        for jt in range(8):
            hs.append(pltpu.make_async_copy(
                tT_hbm.at[pl.ds(8 * jt, 8), pl.ds(c0, 128)],
                inb.at[buf, pl.ds(8 * jt, 8), pl.ds(0, 128)], sem))
        for h in hs:
            h.start()
        return hs

    def wait_in(g, buf, sem):
        ib = blk(g)
        c0 = pl.multiple_of(ib * 128, 128)
        for jt in range(8):
            pltpu.make_async_copy(
                tT_hbm.at[pl.ds(8 * jt, 8), pl.ds(c0, 128)],
                inb.at[buf, pl.ds(8 * jt, 8), pl.ds(0, 128)], sem).wait()

    def out_slice(g):
        r0 = pl.multiple_of(blk(g) * 64, 8)
        return out_hbm.at[pl.ds(r0, 64)]

    j_vecs = [16 * m + it for m in range(4)]

    def transpose_block(buf):
        src = inb.at[buf]

        @plsc.parallel_loop(0, 64, unroll=8)
        def _row(r):
            for lb in range(8):
                p = lb // 4
                cl = jnp.full((16,), 2 * r + p, _i32)
                v = plsc.load_gather(src, [j_vecs[lb % 4], cl])
                outb[buf, r, pl.ds(16 * lb, 16)] = v

    pltpu.make_async_copy(out_hbm.at[pl.ds(0, 64)], outb.at[0], soa).start()
    pltpu.make_async_copy(out_hbm.at[pl.ds(0, 64)], outb.at[1], sob).start()
    fire_in(0, 0, sia)
    fire_in(1, 1, sib)

    def body(k, carry):
        g0 = 2 * k
        wait_in(g0, 0, sia)
        pltpu.make_async_copy(out_hbm.at[pl.ds(0, 64)], outb.at[0], soa).wait()
        transpose_block(0)
        pltpu.make_async_copy(outb.at[0], out_slice(g0), soa).start()
        fire_in(g0 + 2, 0, sia)
        g1 = g0 + 1
        wait_in(g1, 1, sib)
        pltpu.make_async_copy(out_hbm.at[pl.ds(0, 64)], outb.at[1], sob).wait()
        transpose_block(1)
        pltpu.make_async_copy(outb.at[1], out_slice(g1), sob).start()
        fire_in(g1 + 2, 1, sib)
        return carry

    lax.fori_loop(0, _NG1 // 2, body, 0)

    wait_in(_NG1, 0, sia)
    wait_in(_NG1 + 1, 1, sib)
    pltpu.make_async_copy(outb.at[0], out_slice(_NG1 - 2), soa).wait()
    pltpu.make_async_copy(outb.at[1], out_slice(_NG1 - 1), sob).wait()


@functools.partial(
    pl.kernel,
    mesh=_mesh,
    out_type=jax.ShapeDtypeStruct((_HIST, 8, 128, 8, 128), jnp.float32),
    scratch_types=[
        pltpu.VMEM((_BPW,), _i32),
        pltpu.VMEM((_NQ, 128), _i32),
        pltpu.VMEM((8, 128, _D), jnp.float32),
        pltpu.VMEM((4, _D, 131), jnp.float32),
        pltpu.SemaphoreType.DMA,
    ]
    + [pltpu.SemaphoreType.DMA] * 8
    + [pltpu.SemaphoreType.DMA] * 4,
    compiler_params=pltpu.CompilerParams(
        use_tc_tiling_on_sc=False, needs_layout_passes=False),
)
def _s2(idx_hbm, table_hbm, out_hbm, idx_v, icols, rows_v, tbuf, sidx, *sems):
    sem_g = sems[:8]
    sem_o = sems[8:]
    wid = lax.axis_index("s") * _NC + lax.axis_index("c")
    base = wid * _BPW
    it = _iota16()

    pltpu.async_copy(idx_hbm.at[pl.ds(base, _BPW)], idx_v, sidx).wait()

    def extract(q, carry):
        b1 = q // _HIST
        h = q % _HIST
        for m in range(8):
            fv = (128 * b1 + 16 * m + it) * _HIST + h
            v = plsc.load_gather(idx_v, [fv])
            icols[q, pl.ds(16 * m, 16)] = v
        return carry

    lax.fori_loop(0, _NQ, extract, 0)

    jrow_vecs = [16 * jb + it for jb in range(4)]

    def group(grp, carry):
        hg = []
        for j8 in range(8):
            q = grp * 8 + j8
            hg.append(pltpu.async_copy(
                table_hbm.at[icols.at[q]], rows_v.at[j8], sem_g[j8]))
        ho = []
        for j8 in range(8):
            q = grp * 8 + j8
            b1 = q // _HIST + 4 * wid
            h = q % _HIST
            tb = j8 % 4
            hg[j8].wait()
            if j8 >= 4:
                for hh in ho[(j8 - 4) * 8:(j8 - 3) * 8]:
                    hh.wait()

            @plsc.parallel_loop(0, 128, unroll=4)
            def _row(l):
                lv = jnp.full((16,), l, _i32)
                for jb in range(4):
                    v = rows_v[j8, l, pl.ds(16 * jb, 16)]
                    plsc.store_scatter(tbuf.at[tb], [jrow_vecs[jb], lv], v)
            for jg in range(8):
                ho.append(pltpu.async_copy(
                    tbuf.at[tb, pl.ds(8 * jg, 8), pl.ds(0, 128)],
                    out_hbm.at[h, jg, b1], sem_o[tb]))
        for hh in ho[4 * 8:]:
            hh.wait()
        return carry

    lax.fori_loop(0, _NGRP, group, 0)


def kernel(input_index, embeds):
    tT = embeds.T
    tail = embeds[_V0:].reshape(32, 128)
    tlin = _s1(tT, tail)
    table = tlin.reshape(_V, _D)
    flat_idx = input_index.reshape(-1).astype(_i32)
    out5 = _s2(flat_idx, table)
    out = out5.transpose(2, 4, 0, 1, 3).reshape(_BATCH, _HIST, _D)
    return out
